# Initial kernel scaffold; baseline (speedup 1.0000x reference)
import jax
import jax.numpy as jnp
from jax.experimental import pallas as pl


def kernel(observations, start_probs, transitions, emissions):
    raise NotImplementedError("write your pallas kernel here")


# speedup vs baseline: 1.4884x; 1.4884x over previous
import jax
import jax.numpy as jnp
from jax import lax
from jax.experimental import pallas as pl
from jax.experimental.pallas import tpu as pltpu

_S = 512
_E = 2048
_BBLK = 32
_II = 8


def _split3(x):
    mask = jnp.int32(-65536)
    hi_f = lax.bitcast_convert_type(
        lax.bitcast_convert_type(x, jnp.int32) & mask, jnp.float32)
    r1 = x - hi_f
    mid_f = lax.bitcast_convert_type(
        lax.bitcast_convert_type(r1, jnp.int32) & mask, jnp.float32)
    lo_f = r1 - mid_f
    return (hi_f.astype(jnp.bfloat16), mid_f.astype(jnp.bfloat16),
            lo_f.astype(jnp.bfloat16))


def _onehot_gather(onehot_bf16, hi_ref, mid_ref, lo_ref):
    def d(p_ref):
        return jax.lax.dot_general(
            onehot_bf16, p_ref[...],
            dimension_numbers=(((1,), (0,)), ((), ())),
            preferred_element_type=jnp.float32,
        )
    return (d(hi_ref) + d(mid_ref)) + d(lo_ref)


def _fwd_body(obs_ref, sp_ref, trans_ref, ehi_ref, emid_ref, elo_ref,
              vout_ref, v_ref):
    t = pl.program_id(1)

    o_col = jnp.transpose(obs_ref[0])
    iota_e = lax.broadcasted_iota(jnp.int32, (_BBLK, _E), 1)
    onehot = (iota_e == o_col).astype(jnp.bfloat16)
    emit_t = _onehot_gather(onehot, ehi_ref, emid_ref, elo_ref)

    @pl.when(t == 0)
    def _():
        v_ref[...] = sp_ref[...] + emit_t

    @pl.when(t > 0)
    def _():
        v = v_ref[...]
        trans = trans_ref[...]
        acc = None
        for ib in range(_S // _II):
            blk = v[:, ib * _II:(ib + 1) * _II]
            tb = trans[ib * _II:(ib + 1) * _II, :]
            cand = blk[:, :, None] + tb[None, :, :]
            m = jnp.max(cand, axis=1)
            acc = m if acc is None else jnp.maximum(acc, m)
        v_ref[...] = acc + emit_t

    vout_ref[0] = v_ref[...]


def _first_argmax(scores):
    m = jnp.max(scores, axis=1, keepdims=True)
    iota_f = lax.broadcasted_iota(jnp.int32, scores.shape, 1).astype(jnp.float32)
    masked = jnp.where(scores == m, iota_f, jnp.float32(scores.shape[1]))
    return jnp.min(masked, axis=1, keepdims=True).astype(jnp.int32)


def _bwd_body(vall_ref, thi_ref, tmid_ref, tlo_ref, path_ref, state_ref):
    t = pl.program_id(1)
    vt = vall_ref[0]

    @pl.when(t == 0)
    def _():
        state_ref[...] = _first_argmax(vt)

    @pl.when(t > 0)
    def _():
        st = state_ref[...]
        iota_s = lax.broadcasted_iota(jnp.int32, (_BBLK, _S), 1)
        onehot = (iota_s == st).astype(jnp.bfloat16)
        g = _onehot_gather(onehot, thi_ref, tmid_ref, tlo_ref)
        scores = vt + g
        state_ref[...] = _first_argmax(scores)

    path_ref[0] = jnp.transpose(state_ref[...])


def kernel(observations, start_probs, transitions, emissions):
    B, T = observations.shape
    S = transitions.shape[0]
    E = emissions.shape[1]

    obs = observations.astype(jnp.int32)
    obs3 = jnp.transpose(obs).reshape(T, 2, _BBLK).reshape(2 * T, 1, _BBLK)
    sp = start_probs.reshape(1, S)
    emisT = jnp.transpose(emissions)
    transT = jnp.transpose(transitions)
    ehi, emid, elo = _split3(emisT)
    thi, tmid, tlo = _split3(transT)

    v_all = pl.pallas_call(
        _fwd_body,
        grid=(2, T),
        in_specs=[
            pl.BlockSpec((1, 1, _BBLK), lambda c, t: (2 * t + c, 0, 0)),
            pl.BlockSpec((1, S), lambda c, t: (0, 0)),
            pl.BlockSpec((S, S), lambda c, t: (0, 0)),
            pl.BlockSpec((E, S), lambda c, t: (0, 0)),
            pl.BlockSpec((E, S), lambda c, t: (0, 0)),
            pl.BlockSpec((E, S), lambda c, t: (0, 0)),
        ],
        out_specs=pl.BlockSpec((1, _BBLK, S), lambda c, t: (t, c, 0)),
        out_shape=jax.ShapeDtypeStruct((T, B, S), jnp.float32),
        scratch_shapes=[pltpu.VMEM((_BBLK, S), jnp.float32)],
        compiler_params=pltpu.CompilerParams(
            dimension_semantics=("parallel", "arbitrary"),
        ),
    )(obs3, sp, transitions, ehi, emid, elo)

    path3 = pl.pallas_call(
        _bwd_body,
        grid=(2, T),
        in_specs=[
            pl.BlockSpec((1, _BBLK, S), lambda c, t: (T - 1 - t, c, 0)),
            pl.BlockSpec((S, S), lambda c, t: (0, 0)),
            pl.BlockSpec((S, S), lambda c, t: (0, 0)),
            pl.BlockSpec((S, S), lambda c, t: (0, 0)),
        ],
        out_specs=pl.BlockSpec((1, 1, _BBLK), lambda c, t: (2 * (T - 1 - t) + c, 0, 0)),
        out_shape=jax.ShapeDtypeStruct((2 * T, 1, _BBLK), jnp.int32),
        scratch_shapes=[pltpu.VMEM((_BBLK, 1), jnp.int32)],
        compiler_params=pltpu.CompilerParams(
            dimension_semantics=("parallel", "arbitrary"),
        ),
    )(v_all, thi, tmid, tlo)

    path = path3.reshape(T, 2, _BBLK).transpose(1, 2, 0).reshape(B, T)
    return path


# speedup vs baseline: 3.0963x; 2.0803x over previous
import jax
import jax.numpy as jnp
from jax import lax
from jax.experimental import pallas as pl
from jax.experimental.pallas import tpu as pltpu

_S = 512
_E = 2048
_BBLK = 32
_II = 8


def _split3(x):
    mask = jnp.int32(-65536)
    hi_f = lax.bitcast_convert_type(
        lax.bitcast_convert_type(x, jnp.int32) & mask, jnp.float32)
    r1 = x - hi_f
    mid_f = lax.bitcast_convert_type(
        lax.bitcast_convert_type(r1, jnp.int32) & mask, jnp.float32)
    lo_f = r1 - mid_f
    return (hi_f.astype(jnp.bfloat16), mid_f.astype(jnp.bfloat16),
            lo_f.astype(jnp.bfloat16))


def _onehot_gather(onehot_bf16, hi_ref, mid_ref, lo_ref):
    def d(p_ref):
        return jax.lax.dot_general(
            onehot_bf16, p_ref[...],
            dimension_numbers=(((1,), (0,)), ((), ())),
            preferred_element_type=jnp.float32,
        )
    return (d(hi_ref) + d(mid_ref)) + d(lo_ref)


def _fwd_body(obs_ref, sp_ref, trans_ref, ehi_ref, emid_ref, elo_ref,
              vout_ref, v_ref):
    t = pl.program_id(1)

    o_col = jnp.transpose(obs_ref[0])
    iota_e = lax.broadcasted_iota(jnp.int32, (_BBLK, _E), 1)
    onehot = (iota_e == o_col).astype(jnp.bfloat16)
    emit_t = _onehot_gather(onehot, ehi_ref, emid_ref, elo_ref)

    @pl.when(t == 0)
    def _():
        v_ref[...] = sp_ref[...] + emit_t

    @pl.when(t > 0)
    def _():
        v = v_ref[...]
        trans = trans_ref[...]
        acc = None
        for i in range(_S):
            c = v[:, i:i + 1] + trans[i:i + 1, :]
            acc = c if acc is None else jnp.maximum(acc, c)
        v_ref[...] = acc + emit_t

    vout_ref[0] = v_ref[...]


def _first_argmax(scores):
    m = jnp.max(scores, axis=1, keepdims=True)
    iota_f = lax.broadcasted_iota(jnp.int32, scores.shape, 1).astype(jnp.float32)
    masked = jnp.where(scores == m, iota_f, jnp.float32(scores.shape[1]))
    return jnp.min(masked, axis=1, keepdims=True).astype(jnp.int32)


def _bwd_body(vall_ref, thi_ref, tmid_ref, tlo_ref, path_ref, state_ref):
    t = pl.program_id(1)
    vt = vall_ref[0]

    @pl.when(t == 0)
    def _():
        state_ref[...] = _first_argmax(vt)

    @pl.when(t > 0)
    def _():
        st = state_ref[...]
        iota_s = lax.broadcasted_iota(jnp.int32, (_BBLK, _S), 1)
        onehot = (iota_s == st).astype(jnp.bfloat16)
        g = _onehot_gather(onehot, thi_ref, tmid_ref, tlo_ref)
        scores = vt + g
        state_ref[...] = _first_argmax(scores)

    path_ref[0] = jnp.transpose(state_ref[...])


def kernel(observations, start_probs, transitions, emissions):
    B, T = observations.shape
    S = transitions.shape[0]
    E = emissions.shape[1]

    obs = observations.astype(jnp.int32)
    obs3 = jnp.transpose(obs).reshape(T, 2, _BBLK).reshape(2 * T, 1, _BBLK)
    sp = start_probs.reshape(1, S)
    emisT = jnp.transpose(emissions)
    transT = jnp.transpose(transitions)
    ehi, emid, elo = _split3(emisT)
    thi, tmid, tlo = _split3(transT)

    v_all = pl.pallas_call(
        _fwd_body,
        grid=(2, T),
        in_specs=[
            pl.BlockSpec((1, 1, _BBLK), lambda c, t: (2 * t + c, 0, 0)),
            pl.BlockSpec((1, S), lambda c, t: (0, 0)),
            pl.BlockSpec((S, S), lambda c, t: (0, 0)),
            pl.BlockSpec((E, S), lambda c, t: (0, 0)),
            pl.BlockSpec((E, S), lambda c, t: (0, 0)),
            pl.BlockSpec((E, S), lambda c, t: (0, 0)),
        ],
        out_specs=pl.BlockSpec((1, _BBLK, S), lambda c, t: (t, c, 0)),
        out_shape=jax.ShapeDtypeStruct((T, B, S), jnp.float32),
        scratch_shapes=[pltpu.VMEM((_BBLK, S), jnp.float32)],
        compiler_params=pltpu.CompilerParams(
            dimension_semantics=("parallel", "arbitrary"),
        ),
    )(obs3, sp, transitions, ehi, emid, elo)

    path3 = pl.pallas_call(
        _bwd_body,
        grid=(2, T),
        in_specs=[
            pl.BlockSpec((1, _BBLK, S), lambda c, t: (T - 1 - t, c, 0)),
            pl.BlockSpec((S, S), lambda c, t: (0, 0)),
            pl.BlockSpec((S, S), lambda c, t: (0, 0)),
            pl.BlockSpec((S, S), lambda c, t: (0, 0)),
        ],
        out_specs=pl.BlockSpec((1, 1, _BBLK), lambda c, t: (2 * (T - 1 - t) + c, 0, 0)),
        out_shape=jax.ShapeDtypeStruct((2 * T, 1, _BBLK), jnp.int32),
        scratch_shapes=[pltpu.VMEM((_BBLK, 1), jnp.int32)],
        compiler_params=pltpu.CompilerParams(
            dimension_semantics=("parallel", "arbitrary"),
        ),
    )(v_all, thi, tmid, tlo)

    path = path3.reshape(T, 2, _BBLK).transpose(1, 2, 0).reshape(B, T)
    return path


# speedup vs baseline: 4.1477x; 1.3396x over previous
import jax
import jax.numpy as jnp
from jax import lax
from jax.experimental import pallas as pl
from jax.experimental.pallas import tpu as pltpu

_S = 512
_E = 2048
_B = 64


def _split3(x):
    mask = jnp.int32(-65536)
    hi_f = lax.bitcast_convert_type(
        lax.bitcast_convert_type(x, jnp.int32) & mask, jnp.float32)
    r1 = x - hi_f
    mid_f = lax.bitcast_convert_type(
        lax.bitcast_convert_type(r1, jnp.int32) & mask, jnp.float32)
    lo_f = r1 - mid_f
    return (hi_f.astype(jnp.bfloat16), mid_f.astype(jnp.bfloat16),
            lo_f.astype(jnp.bfloat16))


def _onehot_gather(onehot_bf16, hi_ref, mid_ref, lo_ref):
    def d(p_ref):
        return jax.lax.dot_general(
            onehot_bf16, p_ref[...],
            dimension_numbers=(((1,), (0,)), ((), ())),
            preferred_element_type=jnp.float32,
        )
    return (d(hi_ref) + d(mid_ref)) + d(lo_ref)


def _fwd_body(obs_ref, sp_ref, trans_ref, ehi_ref, emid_ref, elo_ref,
              vout_ref, v_ref):
    t = pl.program_id(0)

    o_col = jnp.transpose(obs_ref[0])
    iota_e = lax.broadcasted_iota(jnp.int32, (_B, _E), 1)
    onehot = (iota_e == o_col).astype(jnp.bfloat16)
    emit_t = _onehot_gather(onehot, ehi_ref, emid_ref, elo_ref)

    @pl.when(t == 0)
    def _():
        v_ref[...] = sp_ref[...] + emit_t

    @pl.when(t > 0)
    def _():
        v = v_ref[...]
        trans = trans_ref[...]
        acc = None
        for i in range(_S):
            c = v[:, i:i + 1] + trans[i:i + 1, :]
            acc = c if acc is None else jnp.maximum(acc, c)
        v_ref[...] = acc + emit_t

    vout_ref[0] = v_ref[...]


def _first_argmax(scores):
    m = jnp.max(scores, axis=1, keepdims=True)
    iota_f = lax.broadcasted_iota(jnp.int32, scores.shape, 1).astype(jnp.float32)
    masked = jnp.where(scores == m, iota_f, jnp.float32(scores.shape[1]))
    return jnp.min(masked, axis=1, keepdims=True).astype(jnp.int32)


def _bwd_body(vall_ref, thi_ref, tmid_ref, tlo_ref, path_ref, state_ref):
    t = pl.program_id(0)
    vt = vall_ref[0]

    @pl.when(t == 0)
    def _():
        state_ref[...] = _first_argmax(vt)

    @pl.when(t > 0)
    def _():
        st = state_ref[...]
        iota_s = lax.broadcasted_iota(jnp.int32, (_B, _S), 1)
        onehot = (iota_s == st).astype(jnp.bfloat16)
        g = _onehot_gather(onehot, thi_ref, tmid_ref, tlo_ref)
        scores = vt + g
        state_ref[...] = _first_argmax(scores)

    path_ref[0] = jnp.transpose(state_ref[...])


def kernel(observations, start_probs, transitions, emissions):
    B, T = observations.shape
    S = transitions.shape[0]
    E = emissions.shape[1]

    obs = observations.astype(jnp.int32)
    obs3 = jnp.transpose(obs).reshape(T, 1, B)
    sp = start_probs.reshape(1, S)
    emisT = jnp.transpose(emissions)
    transT = jnp.transpose(transitions)
    ehi, emid, elo = _split3(emisT)
    thi, tmid, tlo = _split3(transT)

    v_all = pl.pallas_call(
        _fwd_body,
        grid=(T,),
        in_specs=[
            pl.BlockSpec((1, 1, B), lambda t: (t, 0, 0)),
            pl.BlockSpec((1, S), lambda t: (0, 0)),
            pl.BlockSpec((S, S), lambda t: (0, 0)),
            pl.BlockSpec((E, S), lambda t: (0, 0)),
            pl.BlockSpec((E, S), lambda t: (0, 0)),
            pl.BlockSpec((E, S), lambda t: (0, 0)),
        ],
        out_specs=pl.BlockSpec((1, B, S), lambda t: (t, 0, 0)),
        out_shape=jax.ShapeDtypeStruct((T, B, S), jnp.float32),
        scratch_shapes=[pltpu.VMEM((B, S), jnp.float32)],
        compiler_params=pltpu.CompilerParams(
            dimension_semantics=("arbitrary",),
        ),
    )(obs3, sp, transitions, ehi, emid, elo)

    path3 = pl.pallas_call(
        _bwd_body,
        grid=(T,),
        in_specs=[
            pl.BlockSpec((1, B, S), lambda t: (T - 1 - t, 0, 0)),
            pl.BlockSpec((S, S), lambda t: (0, 0)),
            pl.BlockSpec((S, S), lambda t: (0, 0)),
            pl.BlockSpec((S, S), lambda t: (0, 0)),
        ],
        out_specs=pl.BlockSpec((1, 1, B), lambda t: (T - 1 - t, 0, 0)),
        out_shape=jax.ShapeDtypeStruct((T, 1, B), jnp.int32),
        scratch_shapes=[pltpu.VMEM((B, 1), jnp.int32)],
        compiler_params=pltpu.CompilerParams(
            dimension_semantics=("arbitrary",),
        ),
    )(v_all, thi, tmid, tlo)

    path = jnp.transpose(path3.reshape(T, B))
    return path


# speedup vs baseline: 4.2551x; 1.0259x over previous
import jax
import jax.numpy as jnp
from jax import lax
from jax.experimental import pallas as pl
from jax.experimental.pallas import tpu as pltpu

_S = 512
_E = 2048
_B = 64


def _split3(x):
    mask = jnp.int32(-65536)
    hi_f = lax.bitcast_convert_type(
        lax.bitcast_convert_type(x, jnp.int32) & mask, jnp.float32)
    r1 = x - hi_f
    mid_f = lax.bitcast_convert_type(
        lax.bitcast_convert_type(r1, jnp.int32) & mask, jnp.float32)
    lo_f = r1 - mid_f
    return (hi_f.astype(jnp.bfloat16), mid_f.astype(jnp.bfloat16),
            lo_f.astype(jnp.bfloat16))


def _onehot_gather(onehot_bf16, hi_ref, mid_ref, lo_ref):
    def d(p_ref):
        return jax.lax.dot_general(
            onehot_bf16, p_ref[...],
            dimension_numbers=(((1,), (0,)), ((), ())),
            preferred_element_type=jnp.float32,
        )
    return (d(hi_ref) + d(mid_ref)) + d(lo_ref)


def _fwd_body(obs_ref, sp_ref, trans_ref, ehi_ref, emid_ref, elo_ref,
              vout_ref, v_ref):
    t = pl.program_id(0)

    o_col = jnp.transpose(obs_ref[0])
    iota_e = lax.broadcasted_iota(jnp.int32, (_B, _E), 1)
    onehot = (iota_e == o_col).astype(jnp.bfloat16)
    emit_t = _onehot_gather(onehot, ehi_ref, emid_ref, elo_ref)

    @pl.when(t == 0)
    def _():
        v_ref[...] = sp_ref[...] + emit_t

    @pl.when(t > 0)
    def _():
        v = v_ref[...]
        trans = trans_ref[...]
        acc = None
        for i in range(_S):
            c = v[:, i:i + 1] + trans[i:i + 1, :]
            acc = c if acc is None else jnp.maximum(acc, c)
        v_ref[...] = acc + emit_t

    vout_ref[0] = v_ref[...]


def _first_argmax(scores):
    m = jnp.max(scores, axis=1, keepdims=True)
    iota_f = lax.broadcasted_iota(jnp.int32, scores.shape, 1).astype(jnp.float32)
    masked = jnp.where(scores == m, iota_f, jnp.float32(scores.shape[1]))
    return jnp.min(masked, axis=1, keepdims=True).astype(jnp.int32)


_KB = 16


def _bwd_step(vt, state_ref, thi_ref, tmid_ref, tlo_ref):
    st = state_ref[...]
    iota_s = lax.broadcasted_iota(jnp.int32, (_B, _S), 1)
    onehot = (iota_s == st).astype(jnp.bfloat16)
    g = _onehot_gather(onehot, thi_ref, tmid_ref, tlo_ref)
    state_ref[...] = _first_argmax(vt + g)


def _bwd_body(vall_ref, thi_ref, tmid_ref, tlo_ref, path_ref, state_ref):
    k = pl.program_id(0)
    for kk in range(_KB - 1, -1, -1):
        vt = vall_ref[kk]
        if kk == _KB - 1:
            @pl.when(k == 0)
            def _():
                state_ref[...] = _first_argmax(vt)

            @pl.when(k > 0)
            def _():
                _bwd_step(vt, state_ref, thi_ref, tmid_ref, tlo_ref)
        else:
            _bwd_step(vt, state_ref, thi_ref, tmid_ref, tlo_ref)
        path_ref[kk] = jnp.transpose(state_ref[...])


def kernel(observations, start_probs, transitions, emissions):
    B, T = observations.shape
    S = transitions.shape[0]
    E = emissions.shape[1]

    obs = observations.astype(jnp.int32)
    obs3 = jnp.transpose(obs).reshape(T, 1, B)
    sp = start_probs.reshape(1, S)
    emisT = jnp.transpose(emissions)
    transT = jnp.transpose(transitions)
    ehi, emid, elo = _split3(emisT)
    thi, tmid, tlo = _split3(transT)

    v_all = pl.pallas_call(
        _fwd_body,
        grid=(T,),
        in_specs=[
            pl.BlockSpec((1, 1, B), lambda t: (t, 0, 0)),
            pl.BlockSpec((1, S), lambda t: (0, 0)),
            pl.BlockSpec((S, S), lambda t: (0, 0)),
            pl.BlockSpec((E, S), lambda t: (0, 0)),
            pl.BlockSpec((E, S), lambda t: (0, 0)),
            pl.BlockSpec((E, S), lambda t: (0, 0)),
        ],
        out_specs=pl.BlockSpec((1, B, S), lambda t: (t, 0, 0)),
        out_shape=jax.ShapeDtypeStruct((T, B, S), jnp.float32),
        scratch_shapes=[pltpu.VMEM((B, S), jnp.float32)],
        compiler_params=pltpu.CompilerParams(
            dimension_semantics=("arbitrary",),
        ),
    )(obs3, sp, transitions, ehi, emid, elo)

    nk = T // _KB
    path3 = pl.pallas_call(
        _bwd_body,
        grid=(nk,),
        in_specs=[
            pl.BlockSpec((_KB, B, S), lambda k: (nk - 1 - k, 0, 0)),
            pl.BlockSpec((S, S), lambda k: (0, 0)),
            pl.BlockSpec((S, S), lambda k: (0, 0)),
            pl.BlockSpec((S, S), lambda k: (0, 0)),
        ],
        out_specs=pl.BlockSpec((_KB, 1, B), lambda k: (nk - 1 - k, 0, 0)),
        out_shape=jax.ShapeDtypeStruct((T, 1, B), jnp.int32),
        scratch_shapes=[pltpu.VMEM((B, 1), jnp.int32)],
        compiler_params=pltpu.CompilerParams(
            dimension_semantics=("arbitrary",),
        ),
    )(v_all, thi, tmid, tlo)

    path = jnp.transpose(path3.reshape(T, B))
    return path


# speedup vs baseline: 4.5018x; 1.0580x over previous
import jax
import jax.numpy as jnp
from jax import lax
from jax.experimental import pallas as pl
from jax.experimental.pallas import tpu as pltpu

_S = 512
_E = 2048
_B = 64


def _split3(x):
    mask = jnp.int32(-65536)
    hi_f = lax.bitcast_convert_type(
        lax.bitcast_convert_type(x, jnp.int32) & mask, jnp.float32)
    r1 = x - hi_f
    mid_f = lax.bitcast_convert_type(
        lax.bitcast_convert_type(r1, jnp.int32) & mask, jnp.float32)
    lo_f = r1 - mid_f
    return (hi_f.astype(jnp.bfloat16), mid_f.astype(jnp.bfloat16),
            lo_f.astype(jnp.bfloat16))


def _onehot_gather(onehot_bf16, hi_ref, mid_ref, lo_ref):
    def d(p_ref):
        return jax.lax.dot_general(
            onehot_bf16, p_ref[...],
            dimension_numbers=(((1,), (0,)), ((), ())),
            preferred_element_type=jnp.float32,
        )
    return (d(hi_ref) + d(mid_ref)) + d(lo_ref)


_KF = 2


def _emit_at(obs_ref, kk, ehi_ref, emid_ref, elo_ref):
    o_col = jnp.transpose(obs_ref[kk])
    iota_e = lax.broadcasted_iota(jnp.int32, (_B, _E), 1)
    onehot = (iota_e == o_col).astype(jnp.bfloat16)
    return _onehot_gather(onehot, ehi_ref, emid_ref, elo_ref)


def _fwd_step(v_ref, trans_ref, emit_t):
    v = v_ref[...]
    trans = trans_ref[...]
    acc = None
    for i in range(_S):
        c = v[:, i:i + 1] + trans[i:i + 1, :]
        acc = c if acc is None else jnp.maximum(acc, c)
    v_ref[...] = acc + emit_t


def _fwd_body(obs_ref, sp_ref, trans_ref, ehi_ref, emid_ref, elo_ref,
              vout_ref, v_ref):
    k = pl.program_id(0)
    for kk in range(_KF):
        emit_t = _emit_at(obs_ref, kk, ehi_ref, emid_ref, elo_ref)
        if kk == 0:
            @pl.when(k == 0)
            def _():
                v_ref[...] = sp_ref[...] + emit_t

            @pl.when(k > 0)
            def _():
                _fwd_step(v_ref, trans_ref, emit_t)
        else:
            _fwd_step(v_ref, trans_ref, emit_t)
        vout_ref[kk] = v_ref[...]


def _first_argmax(scores):
    m = jnp.max(scores, axis=1, keepdims=True)
    iota_f = lax.broadcasted_iota(jnp.int32, scores.shape, 1).astype(jnp.float32)
    masked = jnp.where(scores == m, iota_f, jnp.float32(scores.shape[1]))
    return jnp.min(masked, axis=1, keepdims=True).astype(jnp.int32)


_KB = 16


def _bwd_step(vt, state_ref, thi_ref, tmid_ref, tlo_ref):
    st = state_ref[...]
    iota_s = lax.broadcasted_iota(jnp.int32, (_B, _S), 1)
    onehot = (iota_s == st).astype(jnp.bfloat16)
    g = _onehot_gather(onehot, thi_ref, tmid_ref, tlo_ref)
    state_ref[...] = _first_argmax(vt + g)


def _bwd_body(vall_ref, thi_ref, tmid_ref, tlo_ref, path_ref, state_ref):
    k = pl.program_id(0)
    for kk in range(_KB - 1, -1, -1):
        vt = vall_ref[kk]
        if kk == _KB - 1:
            @pl.when(k == 0)
            def _():
                state_ref[...] = _first_argmax(vt)

            @pl.when(k > 0)
            def _():
                _bwd_step(vt, state_ref, thi_ref, tmid_ref, tlo_ref)
        else:
            _bwd_step(vt, state_ref, thi_ref, tmid_ref, tlo_ref)
        path_ref[kk] = jnp.transpose(state_ref[...])


def kernel(observations, start_probs, transitions, emissions):
    B, T = observations.shape
    S = transitions.shape[0]
    E = emissions.shape[1]

    obs = observations.astype(jnp.int32)
    obs3 = jnp.transpose(obs).reshape(T, 1, B)
    sp = start_probs.reshape(1, S)
    emisT = jnp.transpose(emissions)
    transT = jnp.transpose(transitions)
    ehi, emid, elo = _split3(emisT)
    thi, tmid, tlo = _split3(transT)

    nf = T // _KF
    v_all = pl.pallas_call(
        _fwd_body,
        grid=(nf,),
        in_specs=[
            pl.BlockSpec((_KF, 1, B), lambda k: (k, 0, 0)),
            pl.BlockSpec((1, S), lambda k: (0, 0)),
            pl.BlockSpec((S, S), lambda k: (0, 0)),
            pl.BlockSpec((E, S), lambda k: (0, 0)),
            pl.BlockSpec((E, S), lambda k: (0, 0)),
            pl.BlockSpec((E, S), lambda k: (0, 0)),
        ],
        out_specs=pl.BlockSpec((_KF, B, S), lambda k: (k, 0, 0)),
        out_shape=jax.ShapeDtypeStruct((T, B, S), jnp.float32),
        scratch_shapes=[pltpu.VMEM((B, S), jnp.float32)],
        compiler_params=pltpu.CompilerParams(
            dimension_semantics=("arbitrary",),
        ),
    )(obs3, sp, transitions, ehi, emid, elo)

    nk = T // _KB
    path3 = pl.pallas_call(
        _bwd_body,
        grid=(nk,),
        in_specs=[
            pl.BlockSpec((_KB, B, S), lambda k: (nk - 1 - k, 0, 0)),
            pl.BlockSpec((S, S), lambda k: (0, 0)),
            pl.BlockSpec((S, S), lambda k: (0, 0)),
            pl.BlockSpec((S, S), lambda k: (0, 0)),
        ],
        out_specs=pl.BlockSpec((_KB, 1, B), lambda k: (nk - 1 - k, 0, 0)),
        out_shape=jax.ShapeDtypeStruct((T, 1, B), jnp.int32),
        scratch_shapes=[pltpu.VMEM((B, 1), jnp.int32)],
        compiler_params=pltpu.CompilerParams(
            dimension_semantics=("arbitrary",),
        ),
    )(v_all, thi, tmid, tlo)

    path = jnp.transpose(path3.reshape(T, B))
    return path


# speedup vs baseline: 4.6784x; 1.0392x over previous
import jax
import jax.numpy as jnp
from jax import lax
from jax.experimental import pallas as pl
from jax.experimental.pallas import tpu as pltpu

_S = 512
_E = 2048
_B = 64


def _split3(x):
    mask = jnp.int32(-65536)
    hi_f = lax.bitcast_convert_type(
        lax.bitcast_convert_type(x, jnp.int32) & mask, jnp.float32)
    r1 = x - hi_f
    mid_f = lax.bitcast_convert_type(
        lax.bitcast_convert_type(r1, jnp.int32) & mask, jnp.float32)
    lo_f = r1 - mid_f
    return (hi_f.astype(jnp.bfloat16), mid_f.astype(jnp.bfloat16),
            lo_f.astype(jnp.bfloat16))


def _onehot_gather(onehot_bf16, hi_ref, mid_ref, lo_ref):
    def d(p_ref):
        return jax.lax.dot_general(
            onehot_bf16, p_ref[...],
            dimension_numbers=(((1,), (0,)), ((), ())),
            preferred_element_type=jnp.float32,
        )
    return (d(hi_ref) + d(mid_ref)) + d(lo_ref)


_KF = 4


def _emit_at(obs_ref, kk, ehi_ref, emid_ref, elo_ref):
    o_col = jnp.transpose(obs_ref[kk])
    iota_e = lax.broadcasted_iota(jnp.int32, (_B, _E), 1)
    onehot = (iota_e == o_col).astype(jnp.bfloat16)
    return _onehot_gather(onehot, ehi_ref, emid_ref, elo_ref)


def _fwd_step(v_ref, trans_ref, emit_t):
    v = v_ref[...]
    trans = trans_ref[...]
    acc = None
    for i in range(_S):
        c = v[:, i:i + 1] + trans[i:i + 1, :]
        acc = c if acc is None else jnp.maximum(acc, c)
    v_ref[...] = acc + emit_t


def _fwd_body(obs_ref, sp_ref, trans_ref, ehi_ref, emid_ref, elo_ref,
              vout_ref, v_ref):
    k = pl.program_id(0)
    for kk in range(_KF):
        emit_t = _emit_at(obs_ref, kk, ehi_ref, emid_ref, elo_ref)
        if kk == 0:
            @pl.when(k == 0)
            def _():
                v_ref[...] = sp_ref[...] + emit_t

            @pl.when(k > 0)
            def _():
                _fwd_step(v_ref, trans_ref, emit_t)
        else:
            _fwd_step(v_ref, trans_ref, emit_t)
        vout_ref[kk] = v_ref[...]


def _first_argmax(scores):
    m = jnp.max(scores, axis=1, keepdims=True)
    iota_f = lax.broadcasted_iota(jnp.int32, scores.shape, 1).astype(jnp.float32)
    masked = jnp.where(scores == m, iota_f, jnp.float32(scores.shape[1]))
    return jnp.min(masked, axis=1, keepdims=True).astype(jnp.int32)


_KB = 16


def _bwd_step(vt, state_ref, thi_ref, tmid_ref, tlo_ref):
    st = state_ref[...]
    iota_s = lax.broadcasted_iota(jnp.int32, (_B, _S), 1)
    onehot = (iota_s == st).astype(jnp.bfloat16)
    g = _onehot_gather(onehot, thi_ref, tmid_ref, tlo_ref)
    state_ref[...] = _first_argmax(vt + g)


def _bwd_body(vall_ref, thi_ref, tmid_ref, tlo_ref, path_ref, state_ref):
    k = pl.program_id(0)
    for kk in range(_KB - 1, -1, -1):
        vt = vall_ref[kk]
        if kk == _KB - 1:
            @pl.when(k == 0)
            def _():
                state_ref[...] = _first_argmax(vt)

            @pl.when(k > 0)
            def _():
                _bwd_step(vt, state_ref, thi_ref, tmid_ref, tlo_ref)
        else:
            _bwd_step(vt, state_ref, thi_ref, tmid_ref, tlo_ref)
        path_ref[kk] = jnp.transpose(state_ref[...])


def kernel(observations, start_probs, transitions, emissions):
    B, T = observations.shape
    S = transitions.shape[0]
    E = emissions.shape[1]

    obs = observations.astype(jnp.int32)
    obs3 = jnp.transpose(obs).reshape(T, 1, B)
    sp = start_probs.reshape(1, S)
    emisT = jnp.transpose(emissions)
    transT = jnp.transpose(transitions)
    ehi, emid, elo = _split3(emisT)
    thi, tmid, tlo = _split3(transT)

    nf = T // _KF
    v_all = pl.pallas_call(
        _fwd_body,
        grid=(nf,),
        in_specs=[
            pl.BlockSpec((_KF, 1, B), lambda k: (k, 0, 0)),
            pl.BlockSpec((1, S), lambda k: (0, 0)),
            pl.BlockSpec((S, S), lambda k: (0, 0)),
            pl.BlockSpec((E, S), lambda k: (0, 0)),
            pl.BlockSpec((E, S), lambda k: (0, 0)),
            pl.BlockSpec((E, S), lambda k: (0, 0)),
        ],
        out_specs=pl.BlockSpec((_KF, B, S), lambda k: (k, 0, 0)),
        out_shape=jax.ShapeDtypeStruct((T, B, S), jnp.float32),
        scratch_shapes=[pltpu.VMEM((B, S), jnp.float32)],
        compiler_params=pltpu.CompilerParams(
            dimension_semantics=("arbitrary",),
        ),
    )(obs3, sp, transitions, ehi, emid, elo)

    nk = T // _KB
    path3 = pl.pallas_call(
        _bwd_body,
        grid=(nk,),
        in_specs=[
            pl.BlockSpec((_KB, B, S), lambda k: (nk - 1 - k, 0, 0)),
            pl.BlockSpec((S, S), lambda k: (0, 0)),
            pl.BlockSpec((S, S), lambda k: (0, 0)),
            pl.BlockSpec((S, S), lambda k: (0, 0)),
        ],
        out_specs=pl.BlockSpec((_KB, 1, B), lambda k: (nk - 1 - k, 0, 0)),
        out_shape=jax.ShapeDtypeStruct((T, 1, B), jnp.int32),
        scratch_shapes=[pltpu.VMEM((B, 1), jnp.int32)],
        compiler_params=pltpu.CompilerParams(
            dimension_semantics=("arbitrary",),
        ),
    )(v_all, thi, tmid, tlo)

    path = jnp.transpose(path3.reshape(T, B))
    return path
